# baseline (speedup 1.0000x reference)
import functools

import jax
import jax.numpy as jnp
from jax import lax
from jax.experimental import pallas as pl
from jax.experimental.pallas import tpu as pltpu
from jax.experimental.pallas import tpu_sc as plsc

_NC = 2
_NS = 16
_NW = _NC * _NS

_CHUNK = 64


def _matmul_body(emb_ref, w_ref, b_ref, m_ref):
    m_ref[...] = lax.dot_general(
        emb_ref[...], w_ref[...],
        dimension_numbers=(((1,), (1,)), ((), ())),
        preferred_element_type=jnp.float32,
        precision=lax.Precision.HIGHEST,
    ) + b_ref[...]


def _compute_table(emb_table, W, b, dp):
    V, H = emb_table.shape
    w_pad = jnp.zeros((dp, H), jnp.float32).at[:V].set(W)
    b_pad = jnp.zeros((1, dp), jnp.float32).at[0, :V].set(b)
    return pl.pallas_call(
        _matmul_body,
        out_shape=jax.ShapeDtypeStruct((V, dp), jnp.float32),
    )(emb_table, w_pad, b_pad)


def _gather_rows(table, flat_ids):
    B = flat_ids.shape[0]
    DP = table.shape[1]
    b_per_w = B // _NW
    n_chunks = b_per_w // _CHUNK
    mesh = plsc.VectorSubcoreMesh(core_axis_name="c", subcore_axis_name="s")

    @functools.partial(
        pl.kernel,
        out_type=jax.ShapeDtypeStruct((B, DP), jnp.float32),
        mesh=mesh,
        scratch_types=[
            pltpu.VMEM((_CHUNK,), jnp.int32),
            pltpu.VMEM((_CHUNK, DP), jnp.float32),
            pltpu.SemaphoreType.DMA,
        ],
    )
    def gather_kernel(table_hbm, idx_hbm, out_hbm, idx_v, rows_v, sem):
        wid = lax.axis_index("s") * _NC + lax.axis_index("c")
        base = wid * b_per_w

        def body(i, carry):
            off = base + i * _CHUNK
            pltpu.sync_copy(idx_hbm.at[pl.ds(off, _CHUNK)], idx_v)
            pltpu.async_copy(table_hbm.at[idx_v], rows_v, sem).wait()
            pltpu.sync_copy(rows_v, out_hbm.at[pl.ds(off, _CHUNK)])
            return carry

        lax.fori_loop(0, n_chunks, body, 0)

    return gather_kernel(table, flat_ids)


def kernel(input_ids, emb_table, W, b):
    batch, seq = input_ids.shape
    vocab = emb_table.shape[0]
    dp = 1024
    table = _compute_table(emb_table, W, b, dp)
    flat = input_ids.reshape(-1).astype(jnp.int32)
    out_pad = _gather_rows(table, flat)
    return out_pad[:, :vocab].reshape(batch, seq, vocab)
